# Initial kernel scaffold; baseline (speedup 1.0000x reference)
import jax
import jax.numpy as jnp
from jax.experimental import pallas as pl


def kernel(x_feature, y_feature, x_tables_v, y_tables_v, x_tables_t, y_tables_t, pe):
    raise NotImplementedError("write your pallas kernel here")


# speedup vs baseline: 7.5508x; 7.5508x over previous
import jax
import jax.numpy as jnp
from jax.experimental import pallas as pl

B, S, H = 64, 512, 768
M = 1024
CS = 96
NSUB = 8


def _tc_body(xf_ref, yf_ref, xv_ref, yv_ref, xt_ref, yt_ref, pe_ref,
             outv_ref, outt_ref):
    iota = jax.lax.broadcasted_iota(jnp.int32, (S, M), 1)
    vparts, tparts = [], []
    for i in range(NSUB):
        ox = (xf_ref[0, :, i][:, None] == iota).astype(jnp.bfloat16)
        oy = (yf_ref[0, :, i][:, None] == iota).astype(jnp.bfloat16)
        pv = (jnp.dot(ox, xv_ref[i], preferred_element_type=jnp.float32)
              + jnp.dot(oy, yv_ref[i], preferred_element_type=jnp.float32))
        pt = (jnp.dot(ox, xt_ref[i], preferred_element_type=jnp.float32)
              + jnp.dot(oy, yt_ref[i], preferred_element_type=jnp.float32))
        vparts.append(pv)
        tparts.append(pt)
    outv_ref[0] = jnp.concatenate(vparts, axis=-1) + pe_ref[...]
    outt_ref[0] = jnp.concatenate(tparts, axis=-1) + pe_ref[...]


def kernel(x_feature, y_feature, x_tables_v, y_tables_v, x_tables_t,
           y_tables_t, pe):
    xv = x_tables_v.astype(jnp.bfloat16)
    yv = y_tables_v.astype(jnp.bfloat16)
    xt = x_tables_t.astype(jnp.bfloat16)
    yt = y_tables_t.astype(jnp.bfloat16)
    pe2 = pe[0]
    tab_spec = pl.BlockSpec((NSUB, M, CS), lambda b: (0, 0, 0))
    outv, outt = pl.pallas_call(
        _tc_body,
        grid=(B,),
        in_specs=[
            pl.BlockSpec((1, S, NSUB), lambda b: (b, 0, 0)),
            pl.BlockSpec((1, S, NSUB), lambda b: (b, 0, 0)),
            tab_spec, tab_spec, tab_spec, tab_spec,
            pl.BlockSpec((S, H), lambda b: (0, 0)),
        ],
        out_specs=[pl.BlockSpec((1, S, H), lambda b: (b, 0, 0)),
                   pl.BlockSpec((1, S, H), lambda b: (b, 0, 0))],
        out_shape=[jax.ShapeDtypeStruct((B, S, H), jnp.float32),
                   jax.ShapeDtypeStruct((B, S, H), jnp.float32)],
    )(x_feature, y_feature, xv, yv, xt, yt, pe2)
    return outv, outt
